# Initial kernel scaffold; baseline (speedup 1.0000x reference)
import jax
import jax.numpy as jnp
from jax.experimental import pallas as pl


def kernel(inputs, table, W1, b1, W2, b2):
    raise NotImplementedError("write your pallas kernel here")


# speedup vs baseline: 9.5532x; 9.5532x over previous
import jax
import jax.numpy as jnp
from jax import lax
from jax.experimental import pallas as pl
from jax.experimental.pallas import tpu as pltpu
from jax.experimental.pallas import tpu_sc as plsc

_VOCAB = 100000
_EMBED = 768
_BATCH = 1024
_SEQ = 500
_HID = 16

_ROW_BLK = 2000


def _proj_body(table_ref, w1_ref, out_ref):
    out_ref[...] = lax.dot_general(
        table_ref[...], w1_ref[...],
        (((1,), (0,)), ((), ())),
        preferred_element_type=jnp.float32,
        precision=lax.Precision.HIGHEST,
    )


def _project(table, w1):
    return pl.pallas_call(
        _proj_body,
        grid=(_VOCAB // _ROW_BLK,),
        in_specs=[
            pl.BlockSpec((_ROW_BLK, _EMBED), lambda i: (i, 0)),
            pl.BlockSpec((_EMBED, _HID), lambda i: (0, 0)),
        ],
        out_specs=pl.BlockSpec((_ROW_BLK, _HID), lambda i: (i, 0)),
        out_shape=jax.ShapeDtypeStruct((_VOCAB, _HID), jnp.float32),
    )(table, w1)


_NC = 2
_NS = 16
_NW = _NC * _NS
_BPW = _BATCH // _NW
_CHUNK = 128
_SEQP = 512
_NCHUNK = _SEQP // _CHUNK


def _sc_body(idx_hbm, proj_hbm, sums_hbm, idx_v, rows_v, sums_v, sem):
    wid = lax.axis_index("s") * _NC + lax.axis_index("c")
    base = wid * _BPW
    pltpu.sync_copy(idx_hbm.at[pl.ds(base * _SEQP, _BPW * _SEQP)], idx_v)

    def row_fn(r, _):
        roff = pl.multiple_of(r * _SEQP, _SEQP)
        descs = [
            pltpu.async_copy(
                proj_hbm.at[idx_v.at[pl.ds(roff + j * _CHUNK, _CHUNK)]],
                rows_v.at[pl.ds(j * _CHUNK, _CHUNK)],
                sem,
            )
            for j in range(_NCHUNK)
        ]
        for d in descs:
            d.wait()

        def acc_fn(i, a):
            return a + rows_v[i, :]

        acc = lax.fori_loop(0, _SEQP, acc_fn,
                            jnp.zeros((_HID,), jnp.float32), unroll=16)
        soff = pl.multiple_of(r * _HID, _HID)
        sums_v[pl.ds(soff, _HID)] = acc
        return 0

    lax.fori_loop(0, _BPW, row_fn, 0)
    pltpu.sync_copy(sums_v, sums_hbm.at[pl.ds(base * _HID, _BPW * _HID)])


def _sc_pool(idx, proj):
    mesh = plsc.VectorSubcoreMesh(core_axis_name="c", subcore_axis_name="s")
    f = pl.kernel(
        _sc_body,
        out_type=jax.ShapeDtypeStruct((_BATCH * _HID,), jnp.float32),
        mesh=mesh,
        scratch_types=[
            pltpu.VMEM((_BPW * _SEQP,), jnp.int32),
            pltpu.VMEM((_SEQP, _HID), jnp.float32),
            pltpu.VMEM((_BPW * _HID,), jnp.float32),
            pltpu.SemaphoreType.DMA,
        ],
        compiler_params=pltpu.CompilerParams(use_tc_tiling_on_sc=False),
    )
    return f(idx, proj)


def _head_body(sums_ref, b1_ref, w2_ref, b2_ref, out_ref):
    h = jnp.maximum(sums_ref[...] * (1.0 / _SEQ) + b1_ref[...], 0.0)
    s = jnp.sum(h * w2_ref[...], axis=1, keepdims=True) + b2_ref[...]
    out_ref[...] = 1.0 / (1.0 + jnp.exp(-s))


def _head(sums, b1, w2, b2):
    return pl.pallas_call(
        _head_body,
        out_shape=jax.ShapeDtypeStruct((_BATCH, 1), jnp.float32),
    )(sums, b1.reshape(1, _HID), w2.reshape(1, _HID), b2.reshape(1, 1))


def kernel(inputs, table, W1, b1, W2, b2):
    proj = _project(table, W1)
    proj_p = jnp.pad(proj, ((0, 8), (0, 0)))
    idx_p = jnp.pad(inputs.astype(jnp.int32), ((0, 0), (0, _SEQP - _SEQ)),
                    constant_values=_VOCAB).reshape(_BATCH * _SEQP)
    sums = _sc_pool(idx_p, proj_p).reshape(_BATCH, _HID)
    return _head(sums, b1, W2[:, 0], b2)


# speedup vs baseline: 14.5354x; 1.5215x over previous
import jax
import jax.numpy as jnp
from jax import lax
from jax.experimental import pallas as pl
from jax.experimental.pallas import tpu as pltpu
from jax.experimental.pallas import tpu_sc as plsc

_VOCAB = 100000
_EMBED = 768
_BATCH = 1024
_SEQ = 500
_HID = 16

_ROW_BLK = 2000


def _proj_body(table_ref, w1_ref, out_ref):
    out_ref[...] = lax.dot_general(
        table_ref[...], w1_ref[...],
        (((1,), (0,)), ((), ())),
        preferred_element_type=jnp.float32,
        precision=lax.Precision.DEFAULT,
    )


def _project(table, w1):
    return pl.pallas_call(
        _proj_body,
        grid=(_VOCAB // _ROW_BLK,),
        in_specs=[
            pl.BlockSpec((_ROW_BLK, _EMBED), lambda i: (i, 0)),
            pl.BlockSpec((_EMBED, _HID), lambda i: (0, 0)),
        ],
        out_specs=pl.BlockSpec((_ROW_BLK, _HID), lambda i: (i, 0)),
        out_shape=jax.ShapeDtypeStruct((_VOCAB, _HID), jnp.float32),
    )(table, w1)


_NC = 2
_NS = 16
_NW = _NC * _NS
_BPW = _BATCH // _NW
_CHUNK = 128
_SEQP = 512
_NCHUNK = _SEQP // _CHUNK


def _sc_body(idx_hbm, proj_hbm, sums_hbm, idx_v, rows_v, sums_v, sem):
    wid = lax.axis_index("s") * _NC + lax.axis_index("c")
    base = wid * _BPW
    pltpu.sync_copy(idx_hbm.at[pl.ds(base * _SEQP, _BPW * _SEQP)], idx_v)

    def row_fn(r, _):
        roff = pl.multiple_of(r * _SEQP, _SEQP)
        descs = [
            pltpu.async_copy(
                proj_hbm.at[idx_v.at[pl.ds(roff + j * _CHUNK, _CHUNK)]],
                rows_v.at[pl.ds(j * _CHUNK, _CHUNK)],
                sem,
            )
            for j in range(_NCHUNK)
        ]
        for d in descs:
            d.wait()

        def acc_fn(i, a):
            return a + rows_v[i, :]

        acc = lax.fori_loop(0, _SEQP, acc_fn,
                            jnp.zeros((_HID,), jnp.float32), unroll=16)
        soff = pl.multiple_of(r * _HID, _HID)
        sums_v[pl.ds(soff, _HID)] = acc
        return 0

    lax.fori_loop(0, _BPW, row_fn, 0)
    pltpu.sync_copy(sums_v, sums_hbm.at[pl.ds(base * _HID, _BPW * _HID)])


def _sc_pool(idx, proj):
    mesh = plsc.VectorSubcoreMesh(core_axis_name="c", subcore_axis_name="s")
    f = pl.kernel(
        _sc_body,
        out_type=jax.ShapeDtypeStruct((_BATCH * _HID,), jnp.float32),
        mesh=mesh,
        scratch_types=[
            pltpu.VMEM((_BPW * _SEQP,), jnp.int32),
            pltpu.VMEM((_SEQP, _HID), jnp.float32),
            pltpu.VMEM((_BPW * _HID,), jnp.float32),
            pltpu.SemaphoreType.DMA,
        ],
        compiler_params=pltpu.CompilerParams(use_tc_tiling_on_sc=False),
    )
    return f(idx, proj)


def _head_body(sums_ref, b1_ref, w2_ref, b2_ref, out_ref):
    h = jnp.maximum(sums_ref[...] * (1.0 / _SEQ) + b1_ref[...], 0.0)
    s = jnp.sum(h * w2_ref[...], axis=1, keepdims=True) + b2_ref[...]
    out_ref[...] = 1.0 / (1.0 + jnp.exp(-s))


def _head(sums, b1, w2, b2):
    return pl.pallas_call(
        _head_body,
        out_shape=jax.ShapeDtypeStruct((_BATCH, 1), jnp.float32),
    )(sums, b1.reshape(1, _HID), w2.reshape(1, _HID), b2.reshape(1, 1))


def kernel(inputs, table, W1, b1, W2, b2):
    proj = _project(table, W1)
    proj_p = jnp.pad(proj, ((0, 8), (0, 0)))
    idx_p = jnp.pad(inputs.astype(jnp.int32), ((0, 0), (0, _SEQP - _SEQ)),
                    constant_values=_VOCAB).reshape(_BATCH * _SEQP)
    sums = _sc_pool(idx_p, proj_p).reshape(_BATCH, _HID)
    return _head(sums, b1, W2[:, 0], b2)
